# Initial kernel scaffold; baseline (speedup 1.0000x reference)
import jax
import jax.numpy as jnp
from jax.experimental import pallas as pl


def kernel(texts, emb_table, lin_w, lin_b):
    raise NotImplementedError("write your pallas kernel here")


# speedup vs baseline: 105.1717x; 105.1717x over previous
import functools

import jax
import jax.numpy as jnp
from jax import lax
from jax.experimental import pallas as pl
from jax.experimental.pallas import tpu as pltpu
from jax.experimental.pallas import tpu_sc as plsc

_VOCAB = 30522
_D = 128
_C = 3
_B = 16384
_L = 200

_BLKV = 1024
_VPAD = 30720

_NC = 2
_NS = 16
_NW = _NC * _NS
_ROWS_PER_W = _B // _NW
_CH = 64
_NCHUNK = _ROWS_PER_W // _CH
_SG = _CH // 16


def _project_body(e_ref, w_ref, b_ref, o_ref):
    acc = lax.dot_general(
        w_ref[...], e_ref[...],
        dimension_numbers=(((1,), (1,)), ((), ())),
        preferred_element_type=jnp.float32,
    )
    o_ref[...] = acc * (1.0 / _L) + b_ref[...][:, 0:1]


def _project(emb_table, w8, b2d):
    return pl.pallas_call(
        _project_body,
        grid=(_VPAD // _BLKV,),
        in_specs=[
            pl.BlockSpec((_BLKV, _D), lambda i: (i, 0)),
            pl.BlockSpec((8, _D), lambda i: (0, 0)),
            pl.BlockSpec((8, _D), lambda i: (0, 0)),
        ],
        out_specs=pl.BlockSpec((8, _BLKV), lambda i: (0, i)),
        out_shape=jax.ShapeDtypeStruct((8, _VPAD), jnp.float32),
    )(emb_table, w8, b2d)


def _sc_bag_body(pt_hbm, tx_hbm, out_hbm, p0, p1, p2, tv, ov0, ov1, ov2,
                 sem0, sem1):
    wid = lax.axis_index("s") * _NC + lax.axis_index("c")
    base_row = wid * _ROWS_PER_W

    ps = (p0, p1, p2)
    for c in range(_C):
        pltpu.sync_copy(pt_hbm.at[pl.ds(c * _VPAD, _VPAD)], ps[c])

    sems = (sem0, sem1)
    chlen = _CH * _L

    def start(ch, buf):
        off = (base_row + ch * _CH) * _L
        return pltpu.async_copy(
            tx_hbm.at[pl.ds(off, chlen)],
            tv.at[pl.ds(buf * chlen, chlen)],
            sems[buf],
        )

    cp = start(0, 0)
    lane = lax.iota(jnp.int32, 16) * _L
    zero = jnp.zeros((16,), jnp.float32)
    ovs = (ov0, ov1, ov2)

    for ch in range(_NCHUNK):
        nxt = start(ch + 1, (ch + 1) % 2) if ch + 1 < _NCHUNK else None
        cp.wait()
        buf = ch % 2
        for sg in range(_SG):
            sgbase = lane + (buf * chlen + sg * 16 * _L)

            def body(l, accs, _sgbase=sgbase):
                a0, a1, a2 = accs
                tok = plsc.load_gather(tv, [_sgbase + l])
                a0 = a0 + plsc.load_gather(p0, [tok])
                a1 = a1 + plsc.load_gather(p1, [tok])
                a2 = a2 + plsc.load_gather(p2, [tok])
                return (a0, a1, a2)

            accs = lax.fori_loop(0, _L, body, (zero, zero, zero))
            col = ch * _CH + sg * 16
            for c in range(_C):
                ovs[c][pl.ds(col, 16)] = accs[c]
        cp = nxt

    for c in range(_C):
        pltpu.sync_copy(ovs[c], out_hbm.at[pl.ds(c * _B + base_row, _ROWS_PER_W)])


@functools.cache
def _sc_bag():
    mesh = plsc.VectorSubcoreMesh(
        core_axis_name="c", subcore_axis_name="s",
        num_cores=_NC, num_subcores=_NS,
    )
    return pl.kernel(
        _sc_bag_body,
        out_type=jax.ShapeDtypeStruct((_C * _B,), jnp.float32),
        mesh=mesh,
        compiler_params=pltpu.CompilerParams(needs_layout_passes=False),
        scratch_types=[
            pltpu.VMEM((_VPAD,), jnp.float32),
            pltpu.VMEM((_VPAD,), jnp.float32),
            pltpu.VMEM((_VPAD,), jnp.float32),
            pltpu.VMEM((2 * _CH * _L,), jnp.int32),
            pltpu.VMEM((_ROWS_PER_W,), jnp.float32),
            pltpu.VMEM((_ROWS_PER_W,), jnp.float32),
            pltpu.VMEM((_ROWS_PER_W,), jnp.float32),
            pltpu.SemaphoreType.DMA,
            pltpu.SemaphoreType.DMA,
        ],
    )


def kernel(texts, emb_table, lin_w, lin_b):
    w8 = jnp.zeros((8, _D), jnp.float32).at[:_C].set(lin_w)
    b8 = jnp.zeros((8,), jnp.float32).at[:_C].set(lin_b * (1.0 / _L))
    b2d = jnp.broadcast_to(b8[:, None], (8, _D))
    pt = _project(emb_table, w8, b2d)
    out = _sc_bag()(pt[:_C].reshape(-1), texts.reshape(-1))
    return out.reshape(_C, _B).T


# speedup vs baseline: 126.2796x; 1.2007x over previous
import functools

import jax
import jax.numpy as jnp
from jax import lax
from jax.experimental import pallas as pl
from jax.experimental.pallas import tpu as pltpu
from jax.experimental.pallas import tpu_sc as plsc

_VOCAB = 30522
_D = 128
_C = 3
_B = 16384
_L = 200

_BLKV = 1024
_VPAD = 30720

_NC = 2
_NS = 16
_NW = _NC * _NS
_ROWS_PER_W = _B // _NW
_CH = 64
_NCHUNK = _ROWS_PER_W // _CH
_SG = _CH // 16


def _project_body(e_ref, w_ref, b_ref, o_ref):
    acc = lax.dot_general(
        w_ref[...], e_ref[...],
        dimension_numbers=(((1,), (1,)), ((), ())),
        preferred_element_type=jnp.float32,
    )
    o_ref[...] = (acc * (1.0 / _L) + b_ref[...][:, 0:1])[: _C]


def _project(emb_table, w8, b2d):
    return pl.pallas_call(
        _project_body,
        grid=(_VPAD // _BLKV,),
        in_specs=[
            pl.BlockSpec((_BLKV, _D), lambda i: (i, 0)),
            pl.BlockSpec((8, _D), lambda i: (0, 0)),
            pl.BlockSpec((8, _D), lambda i: (0, 0)),
        ],
        out_specs=pl.BlockSpec((_C, _BLKV), lambda i: (0, i)),
        out_shape=jax.ShapeDtypeStruct((_C, _VPAD), jnp.float32),
    )(emb_table, w8, b2d)


def _sc_bag_body(pt_hbm, tx_hbm, out_hbm, p0, p1, p2, tv, ov0, ov1, ov2,
                 sem0, sem1):
    wid = lax.axis_index("s") * _NC + lax.axis_index("c")
    base_row = wid * _ROWS_PER_W

    ps = (p0, p1, p2)
    for c in range(_C):
        pltpu.sync_copy(pt_hbm.at[pl.ds(c * _VPAD, _VPAD)], ps[c])

    sems = (sem0, sem1)
    chlen = _CH * _L

    def start(ch, buf):
        off = (base_row + ch * _CH) * _L
        return pltpu.async_copy(
            tx_hbm.at[pl.ds(off, chlen)],
            tv.at[pl.ds(buf * chlen, chlen)],
            sems[buf],
        )

    cp = start(0, 0)
    lane = lax.iota(jnp.int32, 16) * _L
    zero = jnp.zeros((16,), jnp.float32)
    ovs = (ov0, ov1, ov2)

    for ch in range(_NCHUNK):
        nxt = start(ch + 1, (ch + 1) % 2) if ch + 1 < _NCHUNK else None
        cp.wait()
        buf = ch % 2
        for sg in range(_SG):
            sgbase = lane + (buf * chlen + sg * 16 * _L)

            def body(l, accs, _sgbase=sgbase):
                a0, a1, a2 = accs
                tok = plsc.load_gather(tv, [_sgbase + l])
                a0 = a0 + plsc.load_gather(p0, [tok])
                a1 = a1 + plsc.load_gather(p1, [tok])
                a2 = a2 + plsc.load_gather(p2, [tok])
                return (a0, a1, a2)

            accs = lax.fori_loop(0, _L, body, (zero, zero, zero), unroll=8)
            col = ch * _CH + sg * 16
            for c in range(_C):
                ovs[c][pl.ds(col, 16)] = accs[c]
        cp = nxt

    for c in range(_C):
        pltpu.sync_copy(ovs[c], out_hbm.at[pl.ds(c * _B + base_row, _ROWS_PER_W)])


@functools.cache
def _sc_bag():
    mesh = plsc.VectorSubcoreMesh(
        core_axis_name="c", subcore_axis_name="s",
        num_cores=_NC, num_subcores=_NS,
    )
    return pl.kernel(
        _sc_bag_body,
        out_type=jax.ShapeDtypeStruct((_C * _B,), jnp.float32),
        mesh=mesh,
        compiler_params=pltpu.CompilerParams(needs_layout_passes=False),
        scratch_types=[
            pltpu.VMEM((_VPAD,), jnp.float32),
            pltpu.VMEM((_VPAD,), jnp.float32),
            pltpu.VMEM((_VPAD,), jnp.float32),
            pltpu.VMEM((2 * _CH * _L,), jnp.int32),
            pltpu.VMEM((_ROWS_PER_W,), jnp.float32),
            pltpu.VMEM((_ROWS_PER_W,), jnp.float32),
            pltpu.VMEM((_ROWS_PER_W,), jnp.float32),
            pltpu.SemaphoreType.DMA,
            pltpu.SemaphoreType.DMA,
        ],
    )


def kernel(texts, emb_table, lin_w, lin_b):
    w8 = jnp.zeros((8, _D), jnp.float32).at[:_C].set(lin_w)
    b8 = jnp.zeros((8,), jnp.float32).at[:_C].set(lin_b * (1.0 / _L))
    b2d = jnp.broadcast_to(b8[:, None], (8, _D))
    pt = _project(emb_table, w8, b2d)
    out = _sc_bag()(pt.reshape(-1), texts.reshape(-1))
    return out.reshape(_C, _B).T
